# baseline (speedup 1.0000x reference)
import functools

import jax
import jax.numpy as jnp
from jax import lax
from jax.experimental import pallas as pl
from jax.experimental.pallas import tpu as pltpu
from jax.experimental.pallas import tpu_sc as plsc

B = 4
T = 2048
D = 64
N = B * T
NC = 2
NS = 16
NW = NC * NS
PER_W = N // NW
CH = 128
NCH = PER_W // CH
LANES = 16
VPR = D // LANES

def _emb_body(idx_hbm, table_hbm, pos_hbm, out_hbm, idx_v, rows_v, pos_v, sem):
    wid = lax.axis_index("s") * NC + lax.axis_index("c")
    base = wid * PER_W
    tbase = lax.rem(base, T)

    pltpu.sync_copy(idx_hbm.at[wid], idx_v)

    copies = [
        pltpu.async_copy(
            table_hbm.at[idx_v.at[j]], rows_v.at[pl.ds(j * CH, CH)], sem
        )
        for j in range(NCH)
    ]
    pltpu.sync_copy(pos_hbm.at[pl.ds(tbase, PER_W)], pos_v)
    for c in copies:
        c.wait()

    def add_row(r, carry):
        for cvec in range(VPR):
            sl = pl.ds(cvec * LANES, LANES)
            rows_v[r, sl] = rows_v[r, sl] + pos_v[r, sl]
        return carry

    lax.fori_loop(0, PER_W, add_row, 0)

    pltpu.sync_copy(rows_v, out_hbm.at[pl.ds(base, PER_W)])


@functools.cache
def _emb_kernel():
    mesh = plsc.VectorSubcoreMesh(core_axis_name="c", subcore_axis_name="s")
    return pl.kernel(
        _emb_body,
        mesh=mesh,
        compiler_params=pltpu.CompilerParams(use_tc_tiling_on_sc=False),
        out_type=jax.ShapeDtypeStruct((N, D), jnp.float32),
        scratch_types=[
            pltpu.VMEM((NCH, CH), jnp.int32),
            pltpu.VMEM((PER_W, D), jnp.float32),
            pltpu.VMEM((PER_W, D), jnp.float32),
            pltpu.SemaphoreType.DMA,
        ],
    )


def kernel(x, embedding_table, pos_table):
    idx = x.astype(jnp.int32).reshape(NW, NCH, CH)
    out = _emb_kernel()(idx, embedding_table, pos_table)
    return out.reshape(B, T, D)
